# baseline (speedup 1.0000x reference)
import functools

import jax
import jax.numpy as jnp
from jax import lax
from jax.experimental import pallas as pl
from jax.experimental.pallas import tpu as pltpu
from jax.experimental.pallas import tpu_sc as plsc

BATCH = 16384
HIST = 200
EMB_DIM = 32

NUM_WORKERS = 32
B_PER_W = BATCH // NUM_WORKERS
CHUNK = B_PER_W
N_CHUNKS = HIST
NBUF = 4
K = 3
NSTG = 2
LANES = 16
assert N_CHUNKS % NBUF == 0 and 0 < K < NBUF and NBUF % NSTG == 0


def _emb_kernel(idx_hbm, table_hbm, out_hbm, *scratch):
    idx_v = scratch[:NBUF]
    rows_v = scratch[NBUF : 2 * NBUF]
    stg_v = scratch[2 * NBUF : 2 * NBUF + NSTG]
    isem, gsem, ssem = scratch[2 * NBUF + NSTG :]

    wid = lax.axis_index("s") * 2 + lax.axis_index("c")
    b0 = wid * B_PER_W
    iota = lax.iota(jnp.int32, LANES)
    perms = [
        ((iota + d) & (LANES - 1)) + eo * LANES
        for eo in range(EMB_DIM // LANES)
        for d in range(LANES)
    ]

    def idx_start(i, b):
        pltpu.async_copy(
            idx_hbm.at[i, pl.ds(b0, CHUNK)], idx_v[b], isem.at[b]
        )

    def idx_wait(b):
        pltpu.make_async_copy(
            idx_hbm.at[0, pl.ds(b0, CHUNK)], idx_v[b], isem.at[b]
        ).wait()

    def gather_start(b):
        pltpu.async_copy(table_hbm.at[idx_v[b]], rows_v[b], gsem.at[b])

    def gather_wait(b):
        pltpu.make_async_copy(
            table_hbm.at[idx_v[b]], rows_v[b], gsem.at[b]
        ).wait()

    def store_start(j, s):
        pltpu.async_copy(
            stg_v[s], out_hbm.at[j, :, pl.ds(b0, CHUNK)], ssem.at[s]
        )

    def store_wait(s):
        pltpu.make_async_copy(
            stg_v[s], out_hbm.at[0, :, pl.ds(b0, CHUNK)], ssem.at[s]
        ).wait()

    def transpose(b1, s):
        rows, stg = rows_v[b1], stg_v[s]

        @pl.loop(0, CHUNK // LANES)
        def _t(rb):
            row_vec = iota + rb * LANES
            for e_vec in perms:
                vals = plsc.load_gather(rows, [row_vec, e_vec])
                plsc.store_scatter(stg, [e_vec, row_vec], vals)

    def finish(j, b1, s, prefetch, swait):
        gather_wait(b1)
        if swait:
            store_wait(s)
        transpose(b1, s)
        store_start(j, s)
        if prefetch:
            idx_start(j + NBUF, b1)

    for b in range(NBUF):
        idx_start(b, b)

    for i in range(2 * NBUF):
        b = i % NBUF
        idx_wait(b)
        gather_start(b)
        j = i - K
        if j >= 0:
            finish(j, j % NBUF, j % NSTG, prefetch=True, swait=j >= NSTG)

    @pl.loop(2 * NBUF, N_CHUNKS - NBUF, step=NBUF)
    def _steady(g):
        for b in range(NBUF):
            i = g + b
            j = i - K
            idx_wait(b)
            gather_start(b)
            finish(j, (b - K) % NBUF, (b - K) % NSTG, True, True)

    for b in range(NBUF):
        i = N_CHUNKS - NBUF + b
        idx_wait(b)
        gather_start(b)
        j = i - K
        finish(j, j % NBUF, j % NSTG, prefetch=j + NBUF < N_CHUNKS, swait=True)

    for j in range(N_CHUNKS - K, N_CHUNKS):
        finish(j, j % NBUF, j % NSTG, prefetch=False, swait=True)
    for s in range(NSTG):
        store_wait(s)


SB = 128


def _xt_kernel(x_hbm, xt_hbm, xin, xout, sem):
    wid = lax.axis_index("s") * 2 + lax.axis_index("c")
    b0 = wid * B_PER_W
    iota = lax.iota(jnp.int32, LANES)
    cperms = [(iota + d) & 7 for d in range(8)]

    for sb in range(B_PER_W // SB):
        pltpu.sync_copy(x_hbm.at[pl.ds(b0 + sb * SB, SB), :], xin)

        @pl.loop(0, SB // LANES)
        def _t(rb):
            row_vec = iota + rb * LANES
            for cb in range(HIST // 8):
                for d in range(8):
                    c_vec = cperms[d] + cb * 8
                    vals = plsc.load_gather(xin, [row_vec, c_vec])
                    plsc.store_scatter(xout, [c_vec, row_vec], vals)

        pltpu.sync_copy(xout, xt_hbm.at[:, pl.ds(b0 + sb * SB, SB)])


@jax.jit
def _transpose_x(x):
    mesh = plsc.VectorSubcoreMesh(core_axis_name="c", subcore_axis_name="s")
    k = functools.partial(
        pl.kernel,
        out_type=jax.ShapeDtypeStruct((HIST, BATCH), jnp.int32),
        mesh=mesh,
        scratch_types=[
            pltpu.VMEM((SB, HIST), jnp.int32),
            pltpu.VMEM((HIST, SB), jnp.int32),
            pltpu.SemaphoreType.DMA,
        ],
        compiler_params=pltpu.CompilerParams(
            use_tc_tiling_on_sc=False, needs_layout_passes=False
        ),
    )(_xt_kernel)
    return k(x)


@jax.jit
def _embedding_lookup(xt, table):
    mesh = plsc.VectorSubcoreMesh(core_axis_name="c", subcore_axis_name="s")
    scratch = (
        [pltpu.VMEM((CHUNK,), jnp.int32) for _ in range(NBUF)]
        + [pltpu.VMEM((CHUNK, EMB_DIM), jnp.float32) for _ in range(NBUF)]
        + [pltpu.VMEM((EMB_DIM, CHUNK), jnp.float32) for _ in range(NSTG)]
        + [
            pltpu.SemaphoreType.DMA((NBUF,)),
            pltpu.SemaphoreType.DMA((NBUF,)),
            pltpu.SemaphoreType.DMA((NSTG,)),
        ]
    )
    k = functools.partial(
        pl.kernel,
        out_type=jax.ShapeDtypeStruct((HIST, EMB_DIM, BATCH), jnp.float32),
        mesh=mesh,
        scratch_types=scratch,
        compiler_params=pltpu.CompilerParams(
            use_tc_tiling_on_sc=False, needs_layout_passes=False
        ),
    )(_emb_kernel)
    return k(xt, table)


def kernel(x, table):
    xt = _transpose_x(x.astype(jnp.int32))
    out_t = _embedding_lookup(xt, table)
    return jnp.transpose(out_t, (2, 0, 1))


# speedup vs baseline: 1.0774x; 1.0774x over previous
import functools

import jax
import jax.numpy as jnp
from jax import lax
from jax.experimental import pallas as pl
from jax.experimental.pallas import tpu as pltpu
from jax.experimental.pallas import tpu_sc as plsc

BATCH = 16384
HIST = 200
EMB_DIM = 32

NUM_WORKERS = 32
B_PER_W = BATCH // NUM_WORKERS
CHUNK = B_PER_W
N_CHUNKS = HIST
NBUF = 4
K = 3
NSTG = 2
NIDX = 8
LANES = 16
assert N_CHUNKS % NIDX == 0 and 0 < K < NBUF and NBUF % NSTG == 0
assert NIDX % NBUF == 0 and NIDX % NSTG == 0


def _emb_kernel(idx_hbm, table_hbm, out_hbm, *scratch):
    idx_v = scratch[:NIDX]
    rows_v = scratch[NIDX : NIDX + NBUF]
    stg_v = scratch[NIDX + NBUF : NIDX + NBUF + NSTG]
    isem, gsem, ssem = scratch[NIDX + NBUF + NSTG :]

    wid = lax.axis_index("s") * 2 + lax.axis_index("c")
    b0 = wid * B_PER_W
    iota = lax.iota(jnp.int32, LANES)
    perms = [
        ((iota + d) & (LANES - 1)) + eo * LANES
        for eo in range(EMB_DIM // LANES)
        for d in range(LANES)
    ]

    def idx_start(i, b):
        pltpu.async_copy(
            idx_hbm.at[i, pl.ds(b0, CHUNK)], idx_v[b], isem.at[b]
        )

    def idx_wait(b):
        pltpu.make_async_copy(
            idx_hbm.at[0, pl.ds(b0, CHUNK)], idx_v[b], isem.at[b]
        ).wait()

    def gather_start(b, ib):
        pltpu.async_copy(table_hbm.at[idx_v[ib]], rows_v[b], gsem.at[b])

    def gather_wait(b, ib):
        pltpu.make_async_copy(
            table_hbm.at[idx_v[ib]], rows_v[b], gsem.at[b]
        ).wait()

    def store_start(j, s):
        pltpu.async_copy(
            stg_v[s], out_hbm.at[j, :, pl.ds(b0, CHUNK)], ssem.at[s]
        )

    def store_wait(s):
        pltpu.make_async_copy(
            stg_v[s], out_hbm.at[0, :, pl.ds(b0, CHUNK)], ssem.at[s]
        ).wait()

    def transpose(b1, s):
        rows, stg = rows_v[b1], stg_v[s]

        @pl.loop(0, CHUNK // LANES)
        def _t(rb):
            row_vec = iota + rb * LANES
            for e_vec in perms:
                vals = plsc.load_gather(rows, [row_vec, e_vec])
                plsc.store_scatter(stg, [e_vec, row_vec], vals)

    def finish(j, b1, jb, s, prefetch, swait):
        gather_wait(b1, jb)
        if swait:
            store_wait(s)
        transpose(b1, s)
        store_start(j, s)
        if prefetch:
            idx_start(j + NIDX, jb)

    for b in range(NIDX):
        idx_start(b, b)

    for i in range(NIDX):
        idx_wait(i % NIDX)
        gather_start(i % NBUF, i % NIDX)
        j = i - K
        if j >= 0:
            finish(j, j % NBUF, j % NIDX, j % NSTG, True, swait=j >= NSTG)

    @pl.loop(NIDX, N_CHUNKS - NIDX, step=NIDX)
    def _steady(g):
        for b in range(NIDX):
            j = g + b - K
            idx_wait(b)
            gather_start(b % NBUF, b)
            finish(j, (b - K) % NBUF, (b - K) % NIDX, (b - K) % NSTG, True, True)

    for b in range(NIDX):
        i = N_CHUNKS - NIDX + b
        idx_wait(b)
        gather_start(b % NBUF, b)
        j = i - K
        finish(
            j, j % NBUF, j % NIDX, j % NSTG,
            prefetch=j + NIDX < N_CHUNKS, swait=True,
        )

    for j in range(N_CHUNKS - K, N_CHUNKS):
        finish(j, j % NBUF, j % NIDX, j % NSTG, prefetch=False, swait=True)
    for s in range(NSTG):
        store_wait(s)


@jax.jit
def _embedding_lookup(xt, table):
    mesh = plsc.VectorSubcoreMesh(core_axis_name="c", subcore_axis_name="s")
    scratch = (
        [pltpu.VMEM((CHUNK,), jnp.int32) for _ in range(NIDX)]
        + [pltpu.VMEM((CHUNK, EMB_DIM), jnp.float32) for _ in range(NBUF)]
        + [pltpu.VMEM((EMB_DIM, CHUNK), jnp.float32) for _ in range(NSTG)]
        + [
            pltpu.SemaphoreType.DMA((NIDX,)),
            pltpu.SemaphoreType.DMA((NBUF,)),
            pltpu.SemaphoreType.DMA((NSTG,)),
        ]
    )
    k = functools.partial(
        pl.kernel,
        out_type=jax.ShapeDtypeStruct((HIST, EMB_DIM, BATCH), jnp.float32),
        mesh=mesh,
        scratch_types=scratch,
        compiler_params=pltpu.CompilerParams(
            use_tc_tiling_on_sc=False, needs_layout_passes=False
        ),
    )(_emb_kernel)
    return k(xt, table)


def kernel(x, table):
    xt = x.T.astype(jnp.int32)
    out_t = _embedding_lookup(xt, table)
    return jnp.transpose(out_t, (2, 0, 1))
